# Initial kernel scaffold; baseline (speedup 1.0000x reference)
import jax
import jax.numpy as jnp
from jax.experimental import pallas as pl


def kernel(x, edge_index, edge_attr, batch, params):
    raise NotImplementedError("write your pallas kernel here")


# speedup vs baseline: 2.8690x; 2.8690x over previous
import functools

import jax
import jax.numpy as jnp
from jax import lax
from jax.experimental import pallas as pl
from jax.experimental.pallas import tpu as pltpu
from jax.experimental.pallas import tpu_sc as plsc

N = 10000
E = 320000
D = 128
DE = 16
G = 128

_NC = 2
_NS = 16
_NW = _NC * _NS
_EPW = E // _NW
_C = 80
_NCH = _EPW // _C
_RPT = 624
_TAIL = N - _NS * _RPT

_HIGH = lax.Precision.HIGHEST


def _dot(a, b):
    return jax.lax.dot_general(a, b, (((1,), (0,)), ((), ())),
                               preferred_element_type=jnp.float32)


def _make_mp():
    mesh = plsc.VectorSubcoreMesh(core_axis_name="c", subcore_axis_name="s")

    @functools.partial(
        pl.kernel,
        mesh=mesh,
        out_type=jax.ShapeDtypeStruct((_NC * N, D), jnp.float32),
        scratch_types=[
            pltpu.VMEM((_C,), jnp.int32),
            pltpu.VMEM((_C,), jnp.int32),
            pltpu.VMEM((_C, D), jnp.float32),
            pltpu.VMEM((_C, D), jnp.float32),
            pltpu.VMEM_SHARED((N, D), jnp.float32),
            pltpu.SemaphoreType.DMA,
        ],
    )
    def mp(h_hbm, e_hbm, src_hbm, dst_hbm, zeros_hbm, out_hbm,
           sidx, didx, rows, erows, agg, sem):
        cid = lax.axis_index("c")
        sid = lax.axis_index("s")
        wid = sid * _NC + cid

        slab = pl.multiple_of(sid * _RPT, 8)
        pltpu.sync_copy(zeros_hbm.at[pl.ds(slab, _RPT)],
                        agg.at[pl.ds(slab, _RPT)])

        @pl.when(sid == 0)
        def _():
            pltpu.sync_copy(zeros_hbm.at[pl.ds(_NS * _RPT, _TAIL)],
                            agg.at[pl.ds(_NS * _RPT, _TAIL)])

        plsc.subcore_barrier()

        base = wid * _EPW

        def chunk(i, carry):
            off = pl.multiple_of(base + i * _C, 8)
            pltpu.sync_copy(src_hbm.at[pl.ds(off, _C)], sidx)
            pltpu.sync_copy(dst_hbm.at[pl.ds(off, _C)], didx)
            gat = pltpu.async_copy(h_hbm.at[sidx], rows, sem)
            pltpu.sync_copy(e_hbm.at[pl.ds(off, _C)], erows)
            gat.wait()

            def rbody(r, c):
                for j in range(D // 16):
                    sl = pl.ds(j * 16, 16)
                    v = rows[r, sl] + erows[r, sl]
                    rows[r, sl] = jnp.maximum(v, 0.0)
                return c

            lax.fori_loop(0, _C, rbody, 0)
            pltpu.sync_copy(rows, agg.at[didx], add=True)
            return carry

        lax.fori_loop(0, _NCH, chunk, 0)
        plsc.subcore_barrier()

        pltpu.sync_copy(agg.at[pl.ds(slab, _RPT)],
                        out_hbm.at[pl.ds(pl.multiple_of(cid * N + slab, 8),
                                         _RPT)])

        @pl.when(sid == 0)
        def _():
            pltpu.sync_copy(
                agg.at[pl.ds(_NS * _RPT, _TAIL)],
                out_hbm.at[pl.ds(pl.multiple_of(cid * N + _NS * _RPT, 8),
                                 _TAIL)])

    return mp


@functools.cache
def _get_mp():
    return _make_mp()


_RN = 400
_GN = N // _RN
_RE = 2000
_GE = E // _RE


def _proj_body(x_ref, w_ref, b_ref, o_ref):
    o_ref[...] = _dot(x_ref[...], w_ref[...]) + b_ref[...]


def _proj(x, w, b):
    return pl.pallas_call(
        _proj_body,
        grid=(_GN,),
        in_specs=[
            pl.BlockSpec((_RN, D), lambda i: (i, 0)),
            pl.BlockSpec((D, D), lambda i: (0, 0)),
            pl.BlockSpec((1, D), lambda i: (0, 0)),
        ],
        out_specs=pl.BlockSpec((_RN, D), lambda i: (i, 0)),
        out_shape=jax.ShapeDtypeStruct((N, D), jnp.float32),
    )(x, w, b.reshape(1, D))


def _edge_body(ea_ref, w_ref, b_ref, o0, o1, o2, o3):
    ea = ea_ref[...]
    outs = (o0, o1, o2, o3)
    for l in range(4):
        outs[l][...] = (_dot(ea, w_ref[l * DE:(l + 1) * DE, :])
                        + b_ref[l:l + 1, :])


def _edge_embed(edge_attr, ws, bs):
    wcat = jnp.concatenate(ws, axis=0)
    bcat = jnp.stack(bs, axis=0)
    return pl.pallas_call(
        _edge_body,
        grid=(_GE,),
        in_specs=[
            pl.BlockSpec((_RE, DE), lambda i: (i, 0)),
            pl.BlockSpec((4 * DE, D), lambda i: (0, 0)),
            pl.BlockSpec((4, D), lambda i: (0, 0)),
        ],
        out_specs=[pl.BlockSpec((_RE, D), lambda i: (i, 0))] * 4,
        out_shape=[jax.ShapeDtypeStruct((E, D), jnp.float32)] * 4,
    )(edge_attr, wcat, bcat)


def _node_body(h_ref, p0_ref, p1_ref, w1, b1, w2, b2, g_ref, bt_ref, o_ref):
    h = h_ref[...]
    u = h + p0_ref[...] + p1_ref[...]
    t = jnp.maximum(_dot(u, w1[...]) + b1[...], 0.0)
    v = jnp.maximum(_dot(t, w2[...]) + b2[...], 0.0)
    hn = h + v
    mu = jnp.mean(hn, axis=-1, keepdims=True)
    var = jnp.mean((hn - mu) ** 2, axis=-1, keepdims=True)
    o_ref[...] = ((hn - mu) * lax.rsqrt(var + 1e-5) * g_ref[...]
                  + bt_ref[...])


def _node_update(h, parts, p):
    return pl.pallas_call(
        _node_body,
        grid=(_GN,),
        in_specs=[
            pl.BlockSpec((_RN, D), lambda i: (i, 0)),
            pl.BlockSpec((_RN, D), lambda i: (i, 0)),
            pl.BlockSpec((_RN, D), lambda i: (i + _GN, 0)),
            pl.BlockSpec((D, D), lambda i: (0, 0)),
            pl.BlockSpec((1, D), lambda i: (0, 0)),
            pl.BlockSpec((D, D), lambda i: (0, 0)),
            pl.BlockSpec((1, D), lambda i: (0, 0)),
            pl.BlockSpec((1, D), lambda i: (0, 0)),
            pl.BlockSpec((1, D), lambda i: (0, 0)),
        ],
        out_specs=pl.BlockSpec((_RN, D), lambda i: (i, 0)),
        out_shape=jax.ShapeDtypeStruct((N, D), jnp.float32),
    )(h, parts, parts, p['W1'], p['b1'].reshape(1, D),
      p['W2'], p['b2'].reshape(1, D),
      p['g'].reshape(1, D), p['bt'].reshape(1, D))


def _pool_body(h_ref, b_ref, w1, b1, w2, b2, w3, b3, o_ref):
    giota = lax.broadcasted_iota(jnp.int32, (1, G), 1)
    ones = jnp.ones((_RN, 1), jnp.float32)

    def body(i, carry):
        s, cnt = carry
        hb = h_ref[pl.ds(i * _RN, _RN), :]
        bb = b_ref[pl.ds(i * _RN, _RN), :]
        onehot = (bb == giota).astype(jnp.float32)
        s = s + jax.lax.dot_general(
            onehot, hb, (((0,), (0,)), ((), ())),
            precision=_HIGH, preferred_element_type=jnp.float32)
        cnt = cnt + jax.lax.dot_general(
            onehot, ones, (((0,), (0,)), ((), ())),
            precision=_HIGH, preferred_element_type=jnp.float32)
        return s, cnt

    s0 = jnp.zeros((G, D), jnp.float32)
    c0 = jnp.zeros((G, 1), jnp.float32)
    s, cnt = lax.fori_loop(0, _GN, body, (s0, c0))
    pooled = s / jnp.maximum(cnt, 1.0)
    o = jnp.maximum(_dot(pooled, w1[...]) + b1[...], 0.0)
    o = jnp.maximum(_dot(o, w2[...]) + b2[...], 0.0)
    o_ref[...] = _dot(o, w3[...]) + b3[...]


def _pool_head(h, batch, p):
    return pl.pallas_call(
        _pool_body,
        out_shape=jax.ShapeDtypeStruct((G, 1), jnp.float32),
    )(h, batch.reshape(N, 1), p['Wf1'], p['bf1'].reshape(1, D),
      p['Wf2'], p['bf2'].reshape(1, D // 2),
      p['Wf3'], p['bf3'].reshape(1, 1))


def kernel(x, edge_index, edge_attr, batch, params):
    src = edge_index[0]
    dst = edge_index[1]
    h = _proj(x, params['Wp'], params['bp'])
    convs = [params['conv%d' % l] for l in range(4)]
    embs = _edge_embed(edge_attr,
                       [c['We'] for c in convs],
                       [c['be'] for c in convs])
    zeros = jnp.zeros((N, D), jnp.float32)
    for l in range(4):
        parts = _get_mp()(h, embs[l], src, dst, zeros)
        h = _node_update(h, parts, convs[l])
    return _pool_head(h, batch, params)
